# baseline (speedup 1.0000x reference)
import functools

import numpy as np
import jax
import jax.numpy as jnp
from jax import lax
from jax.experimental import pallas as pl
from jax.experimental.pallas import tpu as pltpu
from jax.experimental.pallas import tpu_sc as plsc

D_HALF = 64
D_MODEL = 128
LANES = 16
CHUNK = 128

INV_TWO_PI = 0.15915494309189535
MAGIC = 12582912.0

_COS_C = (0.9989871519760831, -19.591110544368195,
          61.597305393820854, -61.089690063946605)
_SIN_C = (6.282446814164697, -41.234040039091646,
          79.18757169991866, -59.246811349574564)

_PERM = np.array([g * 32 + 16 * h + i
                  for g in range(2) for i in range(16) for h in range(2)])


def _sincos_chain(pv, f, bb, a):
    q = pv * f + bb
    t = (q + jnp.float32(MAGIC)) - jnp.float32(MAGIC)
    u = q - t
    y = u * u
    c = jnp.float32(_COS_C[3])
    for k in (2, 1, 0):
        c = c * y + jnp.float32(_COS_C[k])
    s = jnp.float32(_SIN_C[3])
    for k in (2, 1, 0):
        s = s * y + jnp.float32(_SIN_C[k])
    return a * c, a * (s * u)


def _widen(w):
    lo = plsc.bitcast(w << 16, jnp.float32)
    hi = plsc.bitcast(w & jnp.int32(-65536), jnp.float32)
    return lo, hi


def _build(n_tokens, seq_len):
    info = plsc.get_sparse_core_info()
    nc, ns = info.num_cores, info.num_subcores
    nw = nc * ns
    assert n_tokens % (nw * CHUNK) == 0
    per_w = n_tokens // nw
    n_chunks = per_w // CHUNK
    assert n_chunks % 2 == 0

    mesh = plsc.VectorSubcoreMesh(core_axis_name="c", subcore_axis_name="s")
    vm = pltpu.VMEM

    @functools.partial(
        pl.kernel,
        mesh=mesh,
        out_type=jax.ShapeDtypeStruct((n_tokens, D_MODEL), jnp.float32),
        scratch_types=[
            vm((per_w,), jnp.int32),
            vm((2, CHUNK, D_MODEL), jnp.int32),
            vm((2, CHUNK, D_MODEL), jnp.float32),
            pltpu.SemaphoreType.DMA,
            pltpu.SemaphoreType.DMA,
            pltpu.SemaphoreType.DMA,
            pltpu.SemaphoreType.DMA,
        ],
        compiler_params=pltpu.CompilerParams(use_tc_tiling_on_sc=False,
                                             needs_layout_passes=False),
    )
    def kern(x_hbm, tab_hbm, out_hbm,
             idx_v, tab_v, out_v, sem_g0, sem_g1, sem_o0, sem_o1):
        wid = lax.axis_index("s") * nc + lax.axis_index("c")
        base_w = wid * per_w
        sem_g = (sem_g0, sem_g1)
        sem_o = (sem_o0, sem_o1)

        pltpu.sync_copy(x_hbm.at[pl.ds(base_w, per_w)], idx_v)

        def idx_slice(ci):
            return idx_v.at[pl.ds(ci * CHUNK, CHUNK)]

        def start_gather(ci, b):
            pltpu.async_copy(tab_hbm.at[idx_slice(ci)], tab_v.at[b], sem_g[b])

        def wait_gather(ci, b):
            pltpu.make_async_copy(tab_hbm.at[idx_slice(ci)], tab_v.at[b], sem_g[b]).wait()

        def drain_out(ci, b):
            base = base_w + ci * CHUNK
            pltpu.make_async_copy(
                out_v.at[b], out_hbm.at[pl.ds(base, CHUNK)], sem_o[b]).wait()

        start_gather(0, 0)

        def pair_body(cp, carry):
            for b in (0, 1):
                ci = cp * 2 + b

                @pl.when(ci + 1 < n_chunks)
                def _():
                    start_gather(ci + 1, 1 - b)

                @pl.when(ci >= 2)
                def _():
                    drain_out(ci - 2, b)

                wait_gather(ci, b)

                def tok_body(t2, tc):
                    chains = []
                    for u in (0, 1, 2, 3):
                        t = t2 * 4 + u
                        p = lax.rem(base_w + ci * CHUNK + t, seq_len) + 1
                        pv = jnp.full((LANES,), p.astype(jnp.float32))
                        for g in (0, 1):
                            alo, ahi = _widen(tab_v[b, t, pl.ds(D_HALF + g * LANES, LANES)])
                            blo, bhi = _widen(tab_v[b, t, pl.ds(96 + g * LANES, LANES)])
                            for h, (av, bv) in enumerate(((alo, blo), (ahi, bhi))):
                                j = 2 * g + h
                                f = plsc.bitcast(
                                    tab_v[b, t, pl.ds(j * LANES, LANES)], jnp.float32)
                                chains.append((t, j, pv, f, bv, av))
                    results = [(t, j) + _sincos_chain(pv, f, bb, a)
                               for (t, j, pv, f, bb, a) in chains]
                    for t, j, oc, oi in results:
                        out_v[b, t, pl.ds(j * LANES, LANES)] = oc
                        out_v[b, t, pl.ds(D_HALF + j * LANES, LANES)] = oi
                    return tc

                lax.fori_loop(0, CHUNK // 4, tok_body, 0)
                base = base_w + ci * CHUNK
                pltpu.async_copy(out_v.at[b], out_hbm.at[pl.ds(base, CHUNK)], sem_o[b])
            return carry

        lax.fori_loop(0, n_chunks // 2, pair_body, 0)
        drain_out(n_chunks - 2, 0)
        drain_out(n_chunks - 1, 1)

    return kern


def _pack_bf16(table):
    v = table.shape[0]
    pb = table[:, _PERM].astype(jnp.bfloat16)
    return lax.bitcast_convert_type(pb.reshape(v, D_HALF // 2, 2), jnp.int32)


def kernel(x, word_table, freq_table, phase_table):
    b, length = x.shape
    n = b * length
    xf = x.reshape(n)
    scale = jnp.float32(INV_TWO_PI)
    bias_turns = phase_table * scale
    bias_turns = bias_turns - jnp.round(bias_turns)
    merged = jnp.concatenate(
        [lax.bitcast_convert_type(freq_table * scale, jnp.int32),
         _pack_bf16(word_table), _pack_bf16(bias_turns)], axis=1)
    out = _build(n, length)(xf, merged)
    return out.reshape(b, length, D_MODEL)


# speedup vs baseline: 1.5146x; 1.5146x over previous
import functools

import jax
import jax.numpy as jnp
from jax import lax
from jax.experimental import pallas as pl
from jax.experimental.pallas import tpu as pltpu
from jax.experimental.pallas import tpu_sc as plsc

D_HALF = 64
D_MODEL = 128
LANES = 16
CHUNK = 128

INV_TWO_PI = 0.15915494309189535
MAGIC = 12582912.0

_COS_C = (0.9989871519760831, -19.591110544368195,
          61.597305393820854, -61.089690063946605)
_SIN_C = (6.282446814164697, -41.234040039091646,
          79.18757169991866, -59.246811349574564)


def _sincos_chain(pv, f, bb, a):
    q = pv * f + bb
    t = (q + jnp.float32(MAGIC)) - jnp.float32(MAGIC)
    u = q - t
    y = u * u
    c = jnp.float32(_COS_C[3])
    for k in (2, 1, 0):
        c = c * y + jnp.float32(_COS_C[k])
    s = jnp.float32(_SIN_C[3])
    for k in (2, 1, 0):
        s = s * y + jnp.float32(_SIN_C[k])
    return a * c, a * (s * u)


def _build(n_tokens, seq_len):
    info = plsc.get_sparse_core_info()
    nc, ns = info.num_cores, info.num_subcores
    nw = nc * ns
    assert n_tokens % (nw * CHUNK) == 0
    per_w = n_tokens // nw
    n_chunks = per_w // CHUNK
    assert n_chunks % 2 == 0

    mesh = plsc.VectorSubcoreMesh(core_axis_name="c", subcore_axis_name="s")
    vm = pltpu.VMEM

    @functools.partial(
        pl.kernel,
        mesh=mesh,
        out_type=jax.ShapeDtypeStruct((n_tokens, D_MODEL), jnp.float32),
        scratch_types=[
            vm((per_w,), jnp.int32),
            vm((2, CHUNK, 2 * D_MODEL), jnp.float32),
            vm((2, CHUNK, D_MODEL), jnp.float32),
            pltpu.SemaphoreType.DMA,
            pltpu.SemaphoreType.DMA,
            pltpu.SemaphoreType.DMA,
            pltpu.SemaphoreType.DMA,
        ],
        compiler_params=pltpu.CompilerParams(use_tc_tiling_on_sc=True),
    )
    def kern(x_hbm, tab_hbm, out_hbm,
             idx_v, tab_v, out_v, sem_g0, sem_g1, sem_o0, sem_o1):
        wid = lax.axis_index("s") * nc + lax.axis_index("c")
        base_w = wid * per_w
        sem_g = (sem_g0, sem_g1)
        sem_o = (sem_o0, sem_o1)

        pltpu.sync_copy(x_hbm.at[pl.ds(base_w, per_w)], idx_v)

        def idx_slice(ci):
            return idx_v.at[pl.ds(ci * CHUNK, CHUNK)]

        def start_gathers(ci, b):
            pltpu.async_copy(tab_hbm.at[idx_slice(ci)], tab_v.at[b], sem_g[b])

        def wait_gathers(ci, b):
            pltpu.make_async_copy(tab_hbm.at[idx_slice(ci)], tab_v.at[b], sem_g[b]).wait()

        def drain_out(ci, b):
            base = base_w + ci * CHUNK
            pltpu.make_async_copy(
                out_v.at[b], out_hbm.at[pl.ds(base, CHUNK)], sem_o[b]).wait()

        start_gathers(0, 0)

        def pair_body(cp, carry):
            for b in (0, 1):
                ci = cp * 2 + b

                @pl.when(ci + 1 < n_chunks)
                def _():
                    start_gathers(ci + 1, 1 - b)

                @pl.when(ci >= 2)
                def _():
                    drain_out(ci - 2, b)

                wait_gathers(ci, b)

                def tok_body(t2, tc):
                    chains = []
                    for u in (0, 1, 2, 3):
                        t = t2 * 4 + u
                        p = lax.rem(base_w + ci * CHUNK + t, seq_len) + 1
                        pv = jnp.full((LANES,), p.astype(jnp.float32))
                        for j in range(D_HALF // LANES):
                            f = tab_v[b, t, pl.ds(j * LANES, LANES)]
                            a = tab_v[b, t, pl.ds(D_HALF + j * LANES, LANES)]
                            bb = tab_v[b, t, pl.ds(D_MODEL + j * LANES, LANES)]
                            chains.append((t, j, pv, f, bb, a))
                    results = [(t, j) + _sincos_chain(pv, f, bb, a)
                               for (t, j, pv, f, bb, a) in chains]
                    for t, j, oc, oi in results:
                        out_v[b, t, pl.ds(j * LANES, LANES)] = oc
                        out_v[b, t, pl.ds(D_HALF + j * LANES, LANES)] = oi
                    return tc

                lax.fori_loop(0, CHUNK // 4, tok_body, 0)
                base = base_w + ci * CHUNK
                pltpu.async_copy(out_v.at[b], out_hbm.at[pl.ds(base, CHUNK)], sem_o[b])
            return carry

        lax.fori_loop(0, n_chunks // 2, pair_body, 0)
        drain_out(n_chunks - 2, 0)
        drain_out(n_chunks - 1, 1)

    return kern


def kernel(x, word_table, freq_table, phase_table):
    b, length = x.shape
    n = b * length
    v = word_table.shape[0]
    xf = x.reshape(n)
    scale = jnp.float32(INV_TWO_PI)
    merged = jnp.concatenate(
        [freq_table * scale, word_table, phase_table * scale,
         jnp.zeros((v, D_HALF), jnp.float32)], axis=1)
    out = _build(n, length)(xf, merged)
    return out.reshape(b, length, D_MODEL)
